# baseline (speedup 1.0000x reference)
import jax
import jax.numpy as jnp
from jax.experimental import pallas as pl
from jax.experimental.pallas import tpu as pltpu

_HI = 1.03

_BN = 65536


def _mlp_kernel(x_ref, w1_ref, w2_ref, b2_ref, o_ref):
    x = x_ref[...]
    keep = (jnp.max(jnp.abs(x[:3, :]), axis=0, keepdims=True)
            <= _HI).astype(jnp.float32)
    xb = x.astype(jnp.bfloat16)
    ones = jnp.ones((1, xb.shape[1]), jnp.bfloat16)
    x5 = jnp.concatenate([xb, ones], axis=0)
    h = jax.lax.dot_general(w1_ref[...], x5, (((1,), (0,)), ((), ())),
                            preferred_element_type=jnp.float32)
    h = jnp.maximum(h.astype(jnp.bfloat16), 0)
    v = jax.lax.dot_general(w2_ref[...], h, (((1,), (0,)), ((), ())),
                            preferred_element_type=jnp.float32)
    o_ref[...] = (v + b2_ref[...]) * keep


def kernel(xt, W1, b1, W2, b2):
    n = xt.shape[0]
    x_t = xt.T
    w1a = jnp.concatenate([W1.T, b1.reshape(64, 1)], axis=1)
    w1a = w1a.astype(jnp.bfloat16)
    w2t = W2.T.astype(jnp.bfloat16)
    b2t = b2.reshape(3, 1)

    grid = (n // _BN,)
    out_t = pl.pallas_call(
        _mlp_kernel,
        grid=grid,
        in_specs=[
            pl.BlockSpec((4, _BN), lambda i: (0, i)),
            pl.BlockSpec((64, 5), lambda i: (0, 0)),
            pl.BlockSpec((3, 64), lambda i: (0, 0)),
            pl.BlockSpec((3, 1), lambda i: (0, 0)),
        ],
        out_specs=pl.BlockSpec((3, _BN), lambda i: (0, i)),
        out_shape=jax.ShapeDtypeStruct((3, n), xt.dtype),
        compiler_params=pltpu.CompilerParams(
            dimension_semantics=("parallel",),
        ),
    )(x_t, w1a, w2t, b2t)
    return out_t.T
